# Initial kernel scaffold; baseline (speedup 1.0000x reference)
import jax
import jax.numpy as jnp
from jax.experimental import pallas as pl


def kernel(triplets, rel_attrs, ent_table, rel_table, direction_table, rtype_table, source_table, wi):
    raise NotImplementedError("write your pallas kernel here")


# speedup vs baseline: 4.9342x; 4.9342x over previous
import functools

import jax
import jax.numpy as jnp
from jax import lax
from jax.experimental import pallas as pl
from jax.experimental.pallas import tpu as pltpu
from jax.experimental.pallas import tpu_sc as plsc


def _worker_id():
    return lax.axis_index("s") * 2 + lax.axis_index("c")


def _make_phase1(A0, A1, A2, D, NW):
    NCOMB = A0 * A1 * A2
    rows_per_w = NCOMB // NW
    L = 16
    NCH = D // L
    mesh = plsc.VectorSubcoreMesh(core_axis_name="c", subcore_axis_name="s")

    @functools.partial(
        pl.kernel,
        out_type=jax.ShapeDtypeStruct((NCOMB, D), jnp.float32),
        mesh=mesh,
        scratch_types=[
            pltpu.VMEM((A0, D), jnp.float32),
            pltpu.VMEM((A1, D), jnp.float32),
            pltpu.VMEM((A2, D), jnp.float32),
            pltpu.VMEM((4, L), jnp.float32),
            pltpu.VMEM((rows_per_w, D), jnp.float32),
        ],
        compiler_params=pltpu.CompilerParams(needs_layout_passes=False),
    )
    def phase1(dir_hbm, rt_hbm, src_hbm, w_hbm, comb_hbm,
               dir_v, rt_v, src_v, w_v, comb_v):
        pltpu.sync_copy(dir_hbm, dir_v)
        pltpu.sync_copy(rt_hbm, rt_v)
        pltpu.sync_copy(src_hbm, src_v)
        pltpu.sync_copy(w_hbm, w_v)
        base = _worker_id() * rows_per_w
        w1 = w_v[1, :]
        w2 = w_v[2, :]
        w3 = w_v[3, :]
        for i in range(rows_per_w):
            row = base + i
            a0 = row // (A1 * A2)
            rem = row % (A1 * A2)
            a1 = rem // A2
            a2 = rem % A2
            for j in range(NCH):
                sl = pl.ds(j * L, L)
                comb_v[i, sl] = (w1 * dir_v[a0, sl] + w2 * rt_v[a1, sl]
                                 + w3 * src_v[a2, sl])
        pltpu.sync_copy(comb_v, comb_hbm.at[pl.ds(base, rows_per_w)])

    return phase1


def _make_phase2(B, D, NCOMB, NW):
    bpw = B // NW
    CH = 128
    NCHUNK = bpw // CH
    L = 16
    NJ = D // L
    mesh = plsc.VectorSubcoreMesh(core_axis_name="c", subcore_axis_name="s")

    @functools.partial(
        pl.kernel,
        out_type=jax.ShapeDtypeStruct((B,), jnp.float32),
        mesh=mesh,
        scratch_types=[
            pltpu.VMEM((bpw,), jnp.int32),
            pltpu.VMEM((bpw,), jnp.int32),
            pltpu.VMEM((bpw,), jnp.int32),
            pltpu.VMEM((bpw,), jnp.int32),
            pltpu.VMEM((4, L), jnp.float32),
            pltpu.VMEM((CH, D), jnp.float32),
            pltpu.VMEM((CH, D), jnp.float32),
            pltpu.VMEM((CH, D), jnp.float32),
            pltpu.VMEM((CH, D), jnp.float32),
            pltpu.VMEM((bpw,), jnp.float32),
            pltpu.SemaphoreType.DMA,
            pltpu.SemaphoreType.DMA,
            pltpu.SemaphoreType.DMA,
            pltpu.SemaphoreType.DMA,
        ],
        compiler_params=pltpu.CompilerParams(needs_layout_passes=False),
    )
    def phase2(heads_hbm, tails_hbm, rels_hbm, cidx_hbm,
               ent_hbm, rel_hbm, comb_hbm, w_hbm, out_hbm,
               hidx_v, tidx_v, ridx_v, cidx_v, w_v,
               he_v, te_v, re_v, cb_v, out_v,
               sem0, sem1, sem2, sem3):
        base = _worker_id() * bpw
        pltpu.sync_copy(heads_hbm.at[pl.ds(base, bpw)], hidx_v)
        pltpu.sync_copy(tails_hbm.at[pl.ds(base, bpw)], tidx_v)
        pltpu.sync_copy(rels_hbm.at[pl.ds(base, bpw)], ridx_v)
        pltpu.sync_copy(cidx_hbm.at[pl.ds(base, bpw)], cidx_v)
        pltpu.sync_copy(w_hbm, w_v)
        w0 = w_v[0, :]
        lane = lax.iota(jnp.int32, 16)
        lane0 = lane == 0
        for c in range(NCHUNK):
            csl = pl.ds(c * CH, CH)
            c0 = pltpu.async_copy(ent_hbm.at[hidx_v.at[csl]], he_v, sem0)
            c1 = pltpu.async_copy(ent_hbm.at[tidx_v.at[csl]], te_v, sem1)
            c2 = pltpu.async_copy(rel_hbm.at[ridx_v.at[csl]], re_v, sem2)
            c3 = pltpu.async_copy(comb_hbm.at[cidx_v.at[csl]], cb_v, sem3)
            c0.wait()
            c1.wait()
            c2.wait()
            c3.wait()

            def row_body(r, _):
                acc = None
                for j in range(NJ):
                    sl = pl.ds(j * L, L)
                    ht = he_v[r, sl] * te_v[r, sl]
                    re = re_v[r, sl] * w0 + cb_v[r, sl]
                    p = ht * re
                    acc = p if acc is None else acc + p
                s = jnp.sum(acc)
                idxv = jnp.full((16,), c * CH, jnp.int32) + r
                plsc.store_scatter(out_v, [idxv], jnp.full((16,), s),
                                   mask=lane0)
                return 0

            lax.fori_loop(0, CH, row_body, 0)
        pltpu.sync_copy(out_v, out_hbm.at[pl.ds(base, bpw)])

    return phase2


def kernel(triplets, rel_attrs, ent_table, rel_table, direction_table,
           rtype_table, source_table, wi):
    B = triplets.shape[0]
    D = ent_table.shape[1]
    A0 = direction_table.shape[0]
    A1 = rtype_table.shape[0]
    A2 = source_table.shape[0]
    NCOMB = A0 * A1 * A2
    NW = 32

    heads = triplets[:, 0]
    rels = triplets[:, 1]
    tails = triplets[:, 2]
    cidx = (rel_attrs[:, 0] * (A1 * A2) + rel_attrs[:, 1] * A2
            + rel_attrs[:, 2]).astype(jnp.int32)
    wb = jnp.tile(wi.astype(jnp.float32), (1, 16))

    comb = _make_phase1(A0, A1, A2, D, NW)(
        direction_table, rtype_table, source_table, wb)
    out = _make_phase2(B, D, NCOMB, NW)(
        heads, tails, rels, cidx, ent_table, rel_table, comb, wb)
    return out


# speedup vs baseline: 5.1097x; 1.0355x over previous
import functools

import jax
import jax.numpy as jnp
from jax import lax
from jax.experimental import pallas as pl
from jax.experimental.pallas import tpu as pltpu
from jax.experimental.pallas import tpu_sc as plsc


def _worker_id():
    return lax.axis_index("s") * 2 + lax.axis_index("c")


def _make_phase1(A0, A1, A2, D, NW):
    NCOMB = A0 * A1 * A2
    rows_per_w = NCOMB // NW
    L = 16
    NCH = D // L
    mesh = plsc.VectorSubcoreMesh(core_axis_name="c", subcore_axis_name="s")

    @functools.partial(
        pl.kernel,
        out_type=jax.ShapeDtypeStruct((NCOMB, D), jnp.float32),
        mesh=mesh,
        scratch_types=[
            pltpu.VMEM((A0, D), jnp.float32),
            pltpu.VMEM((A1, D), jnp.float32),
            pltpu.VMEM((A2, D), jnp.float32),
            pltpu.VMEM((4, L), jnp.float32),
            pltpu.VMEM((rows_per_w, D), jnp.float32),
        ],
        compiler_params=pltpu.CompilerParams(needs_layout_passes=False),
    )
    def phase1(dir_hbm, rt_hbm, src_hbm, w_hbm, comb_hbm,
               dir_v, rt_v, src_v, w_v, comb_v):
        pltpu.sync_copy(dir_hbm, dir_v)
        pltpu.sync_copy(rt_hbm, rt_v)
        pltpu.sync_copy(src_hbm, src_v)
        pltpu.sync_copy(w_hbm, w_v)
        base = _worker_id() * rows_per_w
        w1 = w_v[1, :]
        w2 = w_v[2, :]
        w3 = w_v[3, :]
        for i in range(rows_per_w):
            row = base + i
            a0 = row // (A1 * A2)
            rem = row % (A1 * A2)
            a1 = rem // A2
            a2 = rem % A2
            for j in range(NCH):
                sl = pl.ds(j * L, L)
                comb_v[i, sl] = (w1 * dir_v[a0, sl] + w2 * rt_v[a1, sl]
                                 + w3 * src_v[a2, sl])
        pltpu.sync_copy(comb_v, comb_hbm.at[pl.ds(base, rows_per_w)])

    return phase1


def _make_phase2(B, D, NCOMB, NW):
    bpw = B // NW
    CH = 64
    NCHUNK = bpw // CH
    L = 16
    NJ = D // L
    mesh = plsc.VectorSubcoreMesh(core_axis_name="c", subcore_axis_name="s")

    @functools.partial(
        pl.kernel,
        out_type=jax.ShapeDtypeStruct((B,), jnp.float32),
        mesh=mesh,
        scratch_types=[
            pltpu.VMEM((bpw,), jnp.int32),
            pltpu.VMEM((bpw,), jnp.int32),
            pltpu.VMEM((bpw,), jnp.int32),
            pltpu.VMEM((bpw,), jnp.int32),
            pltpu.VMEM((4, L), jnp.float32),
            pltpu.VMEM((2, CH, D), jnp.float32),
            pltpu.VMEM((2, CH, D), jnp.float32),
            pltpu.VMEM((2, CH, D), jnp.float32),
            pltpu.VMEM((2, CH, D), jnp.float32),
            pltpu.VMEM((bpw,), jnp.float32),
            [pltpu.SemaphoreType.DMA] * 8,
        ],
        compiler_params=pltpu.CompilerParams(needs_layout_passes=False),
    )
    def phase2(heads_hbm, tails_hbm, rels_hbm, cidx_hbm,
               ent_hbm, rel_hbm, comb_hbm, w_hbm, out_hbm,
               hidx_v, tidx_v, ridx_v, cidx_v, w_v,
               he_v, te_v, re_v, cb_v, out_v, sems):
        base = _worker_id() * bpw
        pltpu.sync_copy(heads_hbm.at[pl.ds(base, bpw)], hidx_v)
        pltpu.sync_copy(tails_hbm.at[pl.ds(base, bpw)], tidx_v)
        pltpu.sync_copy(rels_hbm.at[pl.ds(base, bpw)], ridx_v)
        pltpu.sync_copy(cidx_hbm.at[pl.ds(base, bpw)], cidx_v)
        pltpu.sync_copy(w_hbm, w_v)
        w0 = w_v[0, :]
        lane = lax.iota(jnp.int32, 16)
        lane0 = lane == 0

        pending = {}

        def gather(c):
            slot = c % 2
            csl = pl.ds(c * CH, CH)
            pending[c] = [
                pltpu.async_copy(ent_hbm.at[hidx_v.at[csl]], he_v.at[slot],
                                 sems[4 * slot + 0]),
                pltpu.async_copy(ent_hbm.at[tidx_v.at[csl]], te_v.at[slot],
                                 sems[4 * slot + 1]),
                pltpu.async_copy(rel_hbm.at[ridx_v.at[csl]], re_v.at[slot],
                                 sems[4 * slot + 2]),
                pltpu.async_copy(comb_hbm.at[cidx_v.at[csl]], cb_v.at[slot],
                                 sems[4 * slot + 3]),
            ]

        gather(0)
        for c in range(NCHUNK):
            slot = c % 2
            if c + 1 < NCHUNK:
                gather(c + 1)
            for cp in pending.pop(c):
                cp.wait()

            def row_body(r, _):
                acc = None
                for j in range(NJ):
                    sl = pl.ds(j * L, L)
                    ht = he_v[slot, r, sl] * te_v[slot, r, sl]
                    re = re_v[slot, r, sl] * w0 + cb_v[slot, r, sl]
                    p = ht * re
                    acc = p if acc is None else acc + p
                s = jnp.sum(acc)
                idxv = jnp.full((16,), c * CH, jnp.int32) + r
                plsc.store_scatter(out_v, [idxv], jnp.full((16,), s),
                                   mask=lane0)
                return 0

            lax.fori_loop(0, CH, row_body, 0, unroll=2)
        pltpu.sync_copy(out_v, out_hbm.at[pl.ds(base, bpw)])

    return phase2


def kernel(triplets, rel_attrs, ent_table, rel_table, direction_table,
           rtype_table, source_table, wi):
    B = triplets.shape[0]
    D = ent_table.shape[1]
    A0 = direction_table.shape[0]
    A1 = rtype_table.shape[0]
    A2 = source_table.shape[0]
    NCOMB = A0 * A1 * A2
    NW = 32

    heads = triplets[:, 0]
    rels = triplets[:, 1]
    tails = triplets[:, 2]
    cidx = (rel_attrs[:, 0] * (A1 * A2) + rel_attrs[:, 1] * A2
            + rel_attrs[:, 2]).astype(jnp.int32)
    wb = jnp.tile(wi.astype(jnp.float32), (1, 16))

    comb = _make_phase1(A0, A1, A2, D, NW)(
        direction_table, rtype_table, source_table, wb)
    out = _make_phase2(B, D, NCOMB, NW)(
        heads, tails, rels, cidx, ent_table, rel_table, comb, wb)
    return out


# speedup vs baseline: 6.2818x; 1.2294x over previous
import functools

import jax
import jax.numpy as jnp
from jax import lax
from jax.experimental import pallas as pl
from jax.experimental.pallas import tpu as pltpu
from jax.experimental.pallas import tpu_sc as plsc


def _worker_id():
    return lax.axis_index("s") * 2 + lax.axis_index("c")


def _make_fused(B, D, A0, A1, A2, NW):
    NCOMB = A0 * A1 * A2
    bpw = B // NW
    CH = 64
    NBUF = 3
    NCHUNK = bpw // CH
    L = 16
    NJ = D // L
    NS = 16
    rps = NCOMB // NS
    mesh = plsc.VectorSubcoreMesh(core_axis_name="c", subcore_axis_name="s")

    @functools.partial(
        pl.kernel,
        out_type=jax.ShapeDtypeStruct((B,), jnp.float32),
        mesh=mesh,
        scratch_types=[
            pltpu.VMEM((bpw,), jnp.int32),
            pltpu.VMEM((bpw,), jnp.int32),
            pltpu.VMEM((bpw,), jnp.int32),
            pltpu.VMEM((bpw,), jnp.int32),
            pltpu.VMEM((4, L), jnp.float32),
            pltpu.VMEM((A0, D), jnp.float32),
            pltpu.VMEM((A1, D), jnp.float32),
            pltpu.VMEM((A2, D), jnp.float32),
            pltpu.VMEM((rps, D), jnp.float32),
            pltpu.VMEM_SHARED((NCOMB, D), jnp.float32),
            pltpu.VMEM_SHARED((1000, D), jnp.float32),
            pltpu.VMEM((NBUF, CH, D), jnp.float32),
            pltpu.VMEM((NBUF, CH, D), jnp.float32),
            pltpu.VMEM((NBUF, CH, D), jnp.float32),
            pltpu.VMEM((NBUF, CH, D), jnp.float32),
            pltpu.VMEM((bpw,), jnp.float32),
            [pltpu.SemaphoreType.DMA] * (4 * NBUF),
        ],
        compiler_params=pltpu.CompilerParams(needs_layout_passes=False),
    )
    def fused(heads_hbm, tails_hbm, rels_hbm, cidx_hbm,
              ent_hbm, rel_hbm, dir_hbm, rt_hbm, src_hbm, w_hbm, out_hbm,
              hidx_v, tidx_v, ridx_v, cidx_v, w_v,
              dir_v, rt_v, src_v, combloc_v, comb_sh, rel_sh,
              he_v, te_v, re_v, cb_v, out_v, sems):
        sid = lax.axis_index("s")
        wid = sid * 2 + lax.axis_index("c")
        base = wid * bpw
        pltpu.sync_copy(heads_hbm.at[pl.ds(base, bpw)], hidx_v)
        pltpu.sync_copy(tails_hbm.at[pl.ds(base, bpw)], tidx_v)
        pltpu.sync_copy(rels_hbm.at[pl.ds(base, bpw)], ridx_v)
        pltpu.sync_copy(cidx_hbm.at[pl.ds(base, bpw)], cidx_v)
        pltpu.sync_copy(w_hbm, w_v)
        w0 = w_v[0, :]
        w1 = w_v[1, :]
        w2 = w_v[2, :]
        w3 = w_v[3, :]

        pending = {}

        def gather_ht(c):
            slot = c % NBUF
            csl = pl.ds(c * CH, CH)
            pending[c] = [
                pltpu.async_copy(ent_hbm.at[hidx_v.at[csl]], he_v.at[slot],
                                 sems[4 * slot + 0]),
                pltpu.async_copy(ent_hbm.at[tidx_v.at[csl]], te_v.at[slot],
                                 sems[4 * slot + 1]),
            ]

        def gather_cb(c):
            slot = c % NBUF
            csl = pl.ds(c * CH, CH)
            pending[c].append(
                pltpu.async_copy(rel_sh.at[ridx_v.at[csl]], re_v.at[slot],
                                 sems[4 * slot + 2]))
            pending[c].append(
                pltpu.async_copy(comb_sh.at[cidx_v.at[csl]], cb_v.at[slot],
                                 sems[4 * slot + 3]))

        for c in range(NBUF - 1):
            gather_ht(c)

        @pl.when(sid == 0)
        def _():
            pltpu.sync_copy(rel_hbm, rel_sh)

        pltpu.sync_copy(dir_hbm, dir_v)
        pltpu.sync_copy(rt_hbm, rt_v)
        pltpu.sync_copy(src_hbm, src_v)
        cbase = sid * rps
        for i in range(rps):
            row = cbase + i
            a0 = row // (A1 * A2)
            rem = row % (A1 * A2)
            a1 = rem // A2
            a2 = rem % A2
            for j in range(NJ):
                sl = pl.ds(j * L, L)
                combloc_v[i, sl] = (w1 * dir_v[a0, sl] + w2 * rt_v[a1, sl]
                                    + w3 * src_v[a2, sl])
        pltpu.sync_copy(combloc_v, comb_sh.at[pl.ds(cbase, rps)])
        plsc.subcore_barrier()

        for c in range(NBUF - 1):
            gather_cb(c)

        for c in range(NCHUNK):
            slot = c % NBUF
            if c + NBUF - 1 < NCHUNK:
                cn = c + NBUF - 1
                gather_ht(cn)
                gather_cb(cn)
            for cp in pending.pop(c):
                cp.wait()

            def row_body(r, _):
                acc = None
                for j in range(NJ):
                    sl = pl.ds(j * L, L)
                    ht = he_v[slot, r, sl] * te_v[slot, r, sl]
                    re = re_v[slot, r, sl] * w0 + cb_v[slot, r, sl]
                    p = ht * re
                    acc = p if acc is None else acc + p
                s = jnp.sum(acc)
                idxv = jnp.full((16,), c * CH, jnp.int32) + r
                plsc.store_scatter(out_v, [idxv], jnp.full((16,), s),
                                   mask=lax.iota(jnp.int32, 16) == 0)
                return 0

            lax.fori_loop(0, CH, row_body, 0, unroll=4)
        pltpu.sync_copy(out_v, out_hbm.at[pl.ds(base, bpw)])

    return fused

def kernel(triplets, rel_attrs, ent_table, rel_table, direction_table,
           rtype_table, source_table, wi):
    B = triplets.shape[0]
    D = ent_table.shape[1]
    A0 = direction_table.shape[0]
    A1 = rtype_table.shape[0]
    A2 = source_table.shape[0]
    NW = 32

    heads = triplets[:, 0]
    rels = triplets[:, 1]
    tails = triplets[:, 2]
    cidx = (rel_attrs[:, 0] * (A1 * A2) + rel_attrs[:, 1] * A2
            + rel_attrs[:, 2]).astype(jnp.int32)
    wb = jnp.tile(wi.astype(jnp.float32), (1, 16))

    out = _make_fused(B, D, A0, A1, A2, NW)(
        heads, tails, rels, cidx, ent_table, rel_table,
        direction_table, rtype_table, source_table, wb)
    return out


# speedup vs baseline: 6.2868x; 1.0008x over previous
import functools

import jax
import jax.numpy as jnp
from jax import lax
from jax.experimental import pallas as pl
from jax.experimental.pallas import tpu as pltpu
from jax.experimental.pallas import tpu_sc as plsc


def _worker_id():
    return lax.axis_index("s") * 2 + lax.axis_index("c")


def _make_fused(B, D, A0, A1, A2, NW):
    NCOMB = A0 * A1 * A2
    bpw = B // NW
    CH = 64
    NBUF = 3
    NCHUNK = bpw // CH
    L = 16
    NJ = D // L
    NS = 16
    rps = NCOMB // NS
    mesh = plsc.VectorSubcoreMesh(core_axis_name="c", subcore_axis_name="s")

    @functools.partial(
        pl.kernel,
        out_type=jax.ShapeDtypeStruct((B,), jnp.float32),
        mesh=mesh,
        scratch_types=[
            pltpu.VMEM((bpw,), jnp.int32),
            pltpu.VMEM((bpw,), jnp.int32),
            pltpu.VMEM((bpw,), jnp.int32),
            pltpu.VMEM((bpw,), jnp.int32),
            pltpu.VMEM((4, L), jnp.float32),
            pltpu.VMEM((A0, D), jnp.float32),
            pltpu.VMEM((A1, D), jnp.float32),
            pltpu.VMEM((A2, D), jnp.float32),
            pltpu.VMEM((rps, D), jnp.float32),
            pltpu.VMEM_SHARED((NCOMB, D), jnp.float32),
            pltpu.VMEM_SHARED((1000, D), jnp.float32),
            pltpu.VMEM((NBUF, CH, D), jnp.float32),
            pltpu.VMEM((NBUF, CH, D), jnp.float32),
            pltpu.VMEM((NBUF, CH, D), jnp.float32),
            pltpu.VMEM((NBUF, CH, D), jnp.float32),
            pltpu.VMEM((bpw,), jnp.float32),
            [pltpu.SemaphoreType.DMA] * (4 * NBUF),
        ],
        compiler_params=pltpu.CompilerParams(needs_layout_passes=False),
    )
    def fused(idx4_hbm,
              ent_hbm, rel_hbm, dir_hbm, rt_hbm, src_hbm, w_hbm, out_hbm,
              hidx_v, tidx_v, ridx_v, cidx_v, w_v,
              dir_v, rt_v, src_v, combloc_v, comb_sh, rel_sh,
              he_v, te_v, re_v, cb_v, out_v, sems):
        sid = lax.axis_index("s")
        wid = sid * 2 + lax.axis_index("c")
        base = wid * bpw
        pltpu.sync_copy(idx4_hbm.at[0, pl.ds(base, bpw)], hidx_v)
        pltpu.sync_copy(idx4_hbm.at[1, pl.ds(base, bpw)], tidx_v)
        pltpu.sync_copy(idx4_hbm.at[2, pl.ds(base, bpw)], ridx_v)
        pltpu.sync_copy(idx4_hbm.at[3, pl.ds(base, bpw)], cidx_v)
        pltpu.sync_copy(w_hbm, w_v)
        w0 = w_v[0, :]
        w1 = w_v[1, :]
        w2 = w_v[2, :]
        w3 = w_v[3, :]

        pending = {}

        def gather_ht(c):
            slot = c % NBUF
            csl = pl.ds(c * CH, CH)
            pending[c] = [
                pltpu.async_copy(ent_hbm.at[hidx_v.at[csl]], he_v.at[slot],
                                 sems[4 * slot + 0]),
                pltpu.async_copy(ent_hbm.at[tidx_v.at[csl]], te_v.at[slot],
                                 sems[4 * slot + 1]),
            ]

        def gather_cb(c):
            slot = c % NBUF
            csl = pl.ds(c * CH, CH)
            pending[c].append(
                pltpu.async_copy(rel_sh.at[ridx_v.at[csl]], re_v.at[slot],
                                 sems[4 * slot + 2]))
            pending[c].append(
                pltpu.async_copy(comb_sh.at[cidx_v.at[csl]], cb_v.at[slot],
                                 sems[4 * slot + 3]))

        for c in range(NBUF - 1):
            gather_ht(c)

        @pl.when(sid == 0)
        def _():
            pltpu.sync_copy(rel_hbm, rel_sh)

        pltpu.sync_copy(dir_hbm, dir_v)
        pltpu.sync_copy(rt_hbm, rt_v)
        pltpu.sync_copy(src_hbm, src_v)
        cbase = sid * rps
        for i in range(rps):
            row = cbase + i
            a0 = row // (A1 * A2)
            rem = row % (A1 * A2)
            a1 = rem // A2
            a2 = rem % A2
            for j in range(NJ):
                sl = pl.ds(j * L, L)
                combloc_v[i, sl] = (w1 * dir_v[a0, sl] + w2 * rt_v[a1, sl]
                                    + w3 * src_v[a2, sl])
        pltpu.sync_copy(combloc_v, comb_sh.at[pl.ds(cbase, rps)])
        plsc.subcore_barrier()

        for c in range(NBUF - 1):
            gather_cb(c)

        for c in range(NCHUNK):
            slot = c % NBUF
            if c + NBUF - 1 < NCHUNK:
                cn = c + NBUF - 1
                gather_ht(cn)
                gather_cb(cn)
            for cp in pending.pop(c):
                cp.wait()

            def row_body(r, _):
                acc = None
                for j in range(NJ):
                    sl = pl.ds(j * L, L)
                    ht = he_v[slot, r, sl] * te_v[slot, r, sl]
                    re = re_v[slot, r, sl] * w0 + cb_v[slot, r, sl]
                    p = ht * re
                    acc = p if acc is None else acc + p
                s = jnp.sum(acc)
                idxv = jnp.full((16,), c * CH, jnp.int32) + r
                plsc.store_scatter(out_v, [idxv], jnp.full((16,), s),
                                   mask=lax.iota(jnp.int32, 16) == 0)
                return 0

            lax.fori_loop(0, CH, row_body, 0, unroll=4)
        pltpu.sync_copy(out_v, out_hbm.at[pl.ds(base, bpw)])

    return fused

def kernel(triplets, rel_attrs, ent_table, rel_table, direction_table,
           rtype_table, source_table, wi):
    B = triplets.shape[0]
    D = ent_table.shape[1]
    A0 = direction_table.shape[0]
    A1 = rtype_table.shape[0]
    A2 = source_table.shape[0]
    NW = 32

    cidx = (rel_attrs[:, 0] * (A1 * A2) + rel_attrs[:, 1] * A2
            + rel_attrs[:, 2]).astype(jnp.int32)
    idx4 = jnp.stack([triplets[:, 0], triplets[:, 2], triplets[:, 1], cidx])
    wb = jnp.tile(wi.astype(jnp.float32), (1, 16))

    out = _make_fused(B, D, A0, A1, A2, NW)(
        idx4, ent_table, rel_table,
        direction_table, rtype_table, source_table, wb)
    return out
